# Initial kernel scaffold; baseline (speedup 1.0000x reference)
import jax
import jax.numpy as jnp
from jax.experimental import pallas as pl


def kernel(x, edge_index, batch, W1, b1, W2, b2, W3, b3, Wg1, bg1, Wg2, bg2):
    raise NotImplementedError("write your pallas kernel here")


# speedup vs baseline: 6.9172x; 6.9172x over previous
import functools

import jax
import jax.numpy as jnp
from jax import lax
from jax.experimental import pallas as pl
from jax.experimental.pallas import tpu as pltpu
from jax.experimental.pallas import tpu_sc as plsc

G = 256
R = 1000
B = 80
NTILES = 32
ZRPAD = 640


def _mesh():
    return plsc.VectorSubcoreMesh(core_axis_name="c", subcore_axis_name="s",
                                  num_cores=2, num_subcores=16)


def _dinv_block(degp_blk):
    deg = degp_blk[0, :, 0] + degp_blk[1, :, 0] + 1.0
    return lax.rsqrt(deg)


def _sc_degree(dst, zeros128, ones128, n_nodes, n_edges):
    np_ = ZRPAD * 16
    ept = n_edges // NTILES
    nb = ept // B

    @functools.partial(
        pl.kernel,
        out_type=jax.ShapeDtypeStruct((2, np_, 128), jnp.float32),
        mesh=_mesh(),
        scratch_types=[
            pltpu.VMEM((B,), jnp.int32),
            pltpu.VMEM((B, 128), jnp.float32),
            pltpu.VMEM_SHARED((np_, 128), jnp.float32),
        ],
    )
    def deg_k(dst_hbm, z_hbm, ones_hbm, out_hbm, dstv, onesv, acc):
        cid = lax.axis_index("c")
        sid = lax.axis_index("s")
        wid = sid * 2 + cid
        pltpu.sync_copy(z_hbm.at[pl.ds(sid * ZRPAD, ZRPAD)],
                        acc.at[pl.ds(sid * ZRPAD, ZRPAD)])
        pltpu.sync_copy(ones_hbm, onesv)
        plsc.subcore_barrier()
        base = wid * ept

        def body(i, carry):
            pltpu.sync_copy(dst_hbm.at[pl.ds(base + i * B, B)], dstv)
            pltpu.sync_copy(onesv, acc.at[dstv], add=True)
            return carry

        lax.fori_loop(0, nb, body, 0)
        plsc.subcore_barrier()
        pltpu.sync_copy(acc.at[pl.ds(sid * ZRPAD, ZRPAD)],
                        out_hbm.at[cid, pl.ds(sid * ZRPAD, ZRPAD)])

    return deg_k(dst, zeros128, ones128)


def _sc_aggregate(hp_flat, src, dst, zeros128, n_nodes, n_edges, n_chunks):
    np_ = ZRPAD * 16
    ept = n_edges // NTILES
    nb = ept // B

    @functools.partial(
        pl.kernel,
        out_type=jax.ShapeDtypeStruct((2, n_chunks, np_, 128), jnp.float32),
        mesh=_mesh(),
        scratch_types=[
            pltpu.VMEM((B,), jnp.int32),
            pltpu.VMEM((B,), jnp.int32),
            pltpu.VMEM((B,), jnp.int32),
            pltpu.VMEM((B, 128), jnp.float32),
            pltpu.VMEM_SHARED((np_, 128), jnp.float32),
            pltpu.SemaphoreType.DMA,
        ],
    )
    def agg_k(hp_hbm, src_hbm, dst_hbm, z_hbm, out_hbm,
              srcv, srcsh, dstv, rows, acc, sem):
        cid = lax.axis_index("c")
        sid = lax.axis_index("s")
        wid = sid * 2 + cid
        base = wid * ept

        for c in range(n_chunks):
            pltpu.sync_copy(z_hbm.at[pl.ds(sid * ZRPAD, ZRPAD)],
                            acc.at[pl.ds(sid * ZRPAD, ZRPAD)])
            plsc.subcore_barrier()

            def body(i, carry, c=c):
                off = base + i * B
                pltpu.sync_copy(src_hbm.at[pl.ds(off, B)], srcv)
                pltpu.sync_copy(dst_hbm.at[pl.ds(off, B)], dstv)
                if c == 0:
                    idxref = srcv
                else:
                    for j in range(B // 16):
                        srcsh[pl.ds(j * 16, 16)] = (
                            srcv[pl.ds(j * 16, 16)] + c * n_nodes)
                    idxref = srcsh
                pltpu.async_copy(hp_hbm.at[idxref], rows, sem).wait()
                pltpu.sync_copy(rows, acc.at[dstv], add=True)
                return carry

            lax.fori_loop(0, nb, body, 0)
            plsc.subcore_barrier()
            pltpu.sync_copy(acc.at[pl.ds(sid * ZRPAD, ZRPAD)],
                            out_hbm.at[cid, c, pl.ds(sid * ZRPAD, ZRPAD)])

    return agg_k(hp_flat, src, dst, zeros128)


def _tc_first(x, W1, degp, n_nodes):
    d = x.shape[1]

    def body(x_ref, w_ref, degp_ref, out_ref):
        dinv = _dinv_block(degp_ref[...])
        h = jnp.dot(x_ref[...], w_ref[...],
                    preferred_element_type=jnp.float32)
        out_ref[0] = h * dinv[:, None]

    return pl.pallas_call(
        body,
        grid=(n_nodes // R,),
        in_specs=[
            pl.BlockSpec((R, d), lambda i: (i, 0)),
            pl.BlockSpec((d, d), lambda i: (0, 0)),
            pl.BlockSpec((2, R, 128), lambda i: (0, i, 0)),
        ],
        out_specs=pl.BlockSpec((1, R, d), lambda i: (0, i, 0)),
        out_shape=jax.ShapeDtypeStruct((1, n_nodes, d), jnp.float32),
    )(x, W1, degp)


def _tc_mid(aggp, hpc, degp, b, W, c_in, f_out, n_nodes):
    c_out = f_out // 128
    f_in = c_in * 128

    def body(aggp_ref, hpc_ref, degp_ref, b_ref, w_ref, out_ref):
        dinv = _dinv_block(degp_ref[...])[:, None]
        pieces = []
        for c in range(c_in):
            s = aggp_ref[0, c] + aggp_ref[1, c] + hpc_ref[c]
            pieces.append(
                jnp.maximum(dinv * s + b_ref[0, c * 128:(c + 1) * 128], 0.0))
        xblk = pieces[0] if c_in == 1 else jnp.concatenate(pieces, axis=1)
        h = jnp.dot(xblk, w_ref[...], preferred_element_type=jnp.float32)
        hp = h * dinv
        for c in range(c_out):
            out_ref[c] = hp[:, c * 128:(c + 1) * 128]

    return pl.pallas_call(
        body,
        grid=(n_nodes // R,),
        in_specs=[
            pl.BlockSpec((2, c_in, R, 128), lambda i: (0, 0, i, 0)),
            pl.BlockSpec((c_in, R, 128), lambda i: (0, i, 0)),
            pl.BlockSpec((2, R, 128), lambda i: (0, i, 0)),
            pl.BlockSpec((1, f_in), lambda i: (0, 0)),
            pl.BlockSpec((f_in, f_out), lambda i: (0, 0)),
        ],
        out_specs=pl.BlockSpec((c_out, R, 128), lambda i: (0, i, 0)),
        out_shape=jax.ShapeDtypeStruct((c_out, n_nodes, 128), jnp.float32),
    )(aggp, hpc, degp, b.reshape(1, -1), W)


def _tc_final(aggp3, hpc3, degp, b3, batchr, Wg1, bg1, Wg2, bg2, n_nodes):
    nsteps = n_nodes // R

    def body(aggp_ref, hpc_ref, degp_ref, b_ref, batch_ref,
             wg1_ref, bg1_ref, wg2_ref, bg2_ref, out_ref, acc_ref, y_ref):
        i = pl.program_id(0)

        @pl.when(i == 0)
        def _():
            acc_ref[...] = jnp.full((G, 512), -jnp.inf, jnp.float32)

        dinv = _dinv_block(degp_ref[...])[:, None]
        pieces = []
        for c in range(4):
            s = aggp_ref[0, c] + aggp_ref[1, c] + hpc_ref[c]
            pieces.append(
                jnp.maximum(dinv * s + b_ref[0, c * 128:(c + 1) * 128], 0.0))
        y_ref[...] = jnp.concatenate(pieces, axis=1)

        def rmw(r, carry):
            g = batch_ref[0, 0, r]
            row = y_ref[pl.ds(r, 1), :]
            acc_ref[pl.ds(g, 1), :] = jnp.maximum(acc_ref[pl.ds(g, 1), :], row)
            return carry

        lax.fori_loop(0, R, rmw, 0)

        @pl.when(i == nsteps - 1)
        def _():
            acc = acc_ref[...]
            pooled = jnp.where(acc > -jnp.inf, acc, 0.0)
            z = jnp.maximum(
                jnp.dot(pooled, wg1_ref[...],
                        preferred_element_type=jnp.float32) + bg1_ref[0], 0.0)
            o = jnp.dot(z, wg2_ref[...],
                        preferred_element_type=jnp.float32) + bg2_ref[0]
            out_ref[...] = jnp.maximum(o, 0.0)

    return pl.pallas_call(
        body,
        grid=(nsteps,),
        in_specs=[
            pl.BlockSpec((2, 4, R, 128), lambda i: (0, 0, i, 0)),
            pl.BlockSpec((4, R, 128), lambda i: (0, i, 0)),
            pl.BlockSpec((2, R, 128), lambda i: (0, i, 0)),
            pl.BlockSpec((1, 512), lambda i: (0, 0)),
            pl.BlockSpec((1, 1, R), lambda i: (i, 0, 0),
                         memory_space=pltpu.SMEM),
            pl.BlockSpec((512, 1024), lambda i: (0, 0)),
            pl.BlockSpec((1, 1024), lambda i: (0, 0)),
            pl.BlockSpec((1024, 128), lambda i: (0, 0)),
            pl.BlockSpec((1, 128), lambda i: (0, 0)),
        ],
        out_specs=pl.BlockSpec((G, 128), lambda i: (0, 0)),
        out_shape=jax.ShapeDtypeStruct((G, 128), jnp.float32),
        scratch_shapes=[
            pltpu.VMEM((G, 512), jnp.float32),
            pltpu.VMEM((R, 512), jnp.float32),
        ],
    )(aggp3, hpc3, degp, b3.reshape(1, -1), batchr,
      Wg1, bg1.reshape(1, -1), Wg2, bg2.reshape(1, -1))


def kernel(x, edge_index, batch, W1, b1, W2, b2, W3, b3, Wg1, bg1, Wg2, bg2):
    n_nodes, d = x.shape
    n_edges = edge_index.shape[1]
    src = edge_index[0].astype(jnp.int32)
    dst = edge_index[1].astype(jnp.int32)

    np_ = ZRPAD * 16
    zeros128 = jnp.zeros((np_, 128), jnp.float32)
    ones128 = jnp.ones((B, 128), jnp.float32)

    degp = _sc_degree(dst, zeros128, ones128, n_nodes, n_edges)

    hpc1 = _tc_first(x, W1, degp, n_nodes)
    aggp1 = _sc_aggregate(hpc1.reshape(n_nodes, 128), src, dst,
                          zeros128, n_nodes, n_edges, 1)
    hpc2 = _tc_mid(aggp1, hpc1, degp, b1, W2, 1, 256, n_nodes)
    aggp2 = _sc_aggregate(hpc2.reshape(2 * n_nodes, 128), src, dst,
                          zeros128, n_nodes, n_edges, 2)
    hpc3 = _tc_mid(aggp2, hpc2, degp, b2, W3, 2, 512, n_nodes)
    aggp3 = _sc_aggregate(hpc3.reshape(4 * n_nodes, 128), src, dst,
                          zeros128, n_nodes, n_edges, 4)

    batchr = batch.astype(jnp.int32).reshape(n_nodes // R, 1, R)
    return _tc_final(aggp3, hpc3, degp, b3, batchr,
                     Wg1, bg1, Wg2, bg2, n_nodes)
